# baseline (speedup 1.0000x reference)
import functools

import jax
import jax.numpy as jnp
from jax import lax
from jax.experimental import pallas as pl
from jax.experimental.pallas import tpu as pltpu
from jax.experimental.pallas import tpu_sc as plsc

_B, _H, _W = 16, 2048, 2048
_K = 64
_LANE = 128
_CPR = _W // _LANE
_CH = _B * _H * _CPR
_NW = 32
_SLAB = _CH // _NW
_CAP = 64


def _entry_tables(X, y_idx, x_idx):
    b = jnp.repeat(jnp.arange(_B, dtype=jnp.int32), _K)
    k = jnp.tile(jnp.arange(_K, dtype=jnp.int32), (_B,))
    y = y_idx[k]
    x = x_idx[k]
    chunk = (b * _H + y) * _CPR + x // _LANE
    off = x % _LANE
    val = X.reshape(-1)
    tile = chunk // _SLAB
    order = jnp.argsort(tile, stable=True)
    t_s = tile[order]
    chunk_s = chunk[order]
    off_s = off[order]
    val_s = val[order]
    counts = jnp.sum(t_s[None, :] == jnp.arange(_NW, dtype=jnp.int32)[:, None],
                     axis=1).astype(jnp.int32)
    starts = jnp.concatenate(
        [jnp.zeros((1,), jnp.int32), jnp.cumsum(counts)[:-1].astype(jnp.int32)])
    r = jnp.arange(_B * _K, dtype=jnp.int32) - starts[t_s]
    cand = (jnp.arange(_NW, dtype=jnp.int32)[:, None] * _SLAB
            + jnp.arange(_K + 1, dtype=jnp.int32)[None, :] * _CPR)
    hit = jnp.any(cand[:, :, None] == chunk[None, None, :], axis=-1)
    pad = cand[jnp.arange(_NW), jnp.argmin(hit, axis=1)]
    idx_all = jnp.broadcast_to(pad[:, None], (_NW, _CAP)).at[t_s, r].set(chunk_s)
    off_all = jnp.zeros((_NW, _CAP), jnp.int32).at[t_s, r].set(off_s)
    val_all = jnp.zeros((_NW, _CAP), jnp.float32).at[t_s, r].set(val_s)
    return idx_all, off_all, val_all


_mesh = plsc.VectorSubcoreMesh(core_axis_name="c", subcore_axis_name="s",
                               num_cores=2, num_subcores=16)


@functools.partial(
    pl.kernel,
    out_type=jax.ShapeDtypeStruct((_CH, _LANE), jnp.float32),
    mesh=_mesh,
    compiler_params=pltpu.CompilerParams(needs_layout_passes=False),
    scratch_types=[
        pltpu.VMEM((_CAP,), jnp.int32),
        pltpu.VMEM((_CAP,), jnp.int32),
        pltpu.VMEM((_CAP,), jnp.float32),
        pltpu.VMEM((_CAP, _LANE), jnp.float32),
        pltpu.SemaphoreType.DMA,
        pltpu.SemaphoreType.DMA,
        pltpu.SemaphoreType.DMA,
    ],
)
def _sc_scatter(Y2, idx_hbm, off_hbm, val_hbm, out, idx_v, off_v, val_v,
                chunks_v, sem_b, sem_g, sem_s):
    wid = lax.axis_index("s") * 2 + lax.axis_index("c")
    base = wid * _SLAB
    bulk = pltpu.make_async_copy(Y2.at[pl.ds(base, _SLAB)],
                                 out.at[pl.ds(base, _SLAB)], sem_b)
    bulk.start()
    pltpu.sync_copy(idx_hbm.at[wid], idx_v)
    pltpu.sync_copy(off_hbm.at[wid], off_v)
    pltpu.sync_copy(val_hbm.at[wid], val_v)
    pltpu.async_copy(Y2.at[idx_v], chunks_v, sem_g).wait()
    for g in range(_CAP // 16):
        rows = lax.iota(jnp.int32, 16) + g * 16
        cols = off_v[pl.ds(g * 16, 16)]
        vals = val_v[pl.ds(g * 16, 16)]
        plsc.addupdate_scatter(chunks_v, [rows, cols], vals)
    bulk.wait()
    pltpu.async_copy(chunks_v, out.at[idx_v], sem_s).wait()


def kernel(Y, X, y_idx, x_idx):
    idx_all, off_all, val_all = _entry_tables(X, y_idx, x_idx)
    out = _sc_scatter(Y.reshape(_CH, _LANE), idx_all, off_all, val_all)
    return out.reshape(_B, _H, _W)


# speedup vs baseline: 40.6367x; 40.6367x over previous
import functools

import jax
import jax.numpy as jnp
from jax import lax
from jax.experimental import pallas as pl
from jax.experimental.pallas import tpu as pltpu
from jax.experimental.pallas import tpu_sc as plsc

_B, _H, _W = 16, 2048, 2048
_K = 64
_NW = 32
_EPT = _B * _K // _NW
_RB = 256
_NRB = _H // _RB

_mesh = plsc.VectorSubcoreMesh(core_axis_name="c", subcore_axis_name="s",
                               num_cores=2, num_subcores=16)


@functools.partial(
    pl.kernel,
    out_type=jax.ShapeDtypeStruct((_B * _K, _W), jnp.float32),
    mesh=_mesh,
    compiler_params=pltpu.CompilerParams(needs_layout_passes=False),
    scratch_types=[
        pltpu.VMEM((_EPT,), jnp.int32),
        pltpu.VMEM((_EPT,), jnp.int32),
        pltpu.VMEM((_EPT,), jnp.float32),
        pltpu.VMEM((_EPT, _W), jnp.float32),
        pltpu.SemaphoreType.DMA,
    ],
)
def _sc_patch(Y2, idx_hbm, off_hbm, val_hbm, patch, idx_v, off_v, val_v,
              rows_v, sem_g):
    wid = lax.axis_index("s") * 2 + lax.axis_index("c")
    pltpu.sync_copy(idx_hbm.at[wid], idx_v)
    pltpu.sync_copy(off_hbm.at[wid], off_v)
    pltpu.sync_copy(val_hbm.at[wid], val_v)
    pltpu.async_copy(Y2.at[idx_v], rows_v, sem_g).wait()
    for g in range(_EPT // 16):
        rows = lax.iota(jnp.int32, 16) + g * 16
        cols = off_v[pl.ds(g * 16, 16)]
        vals = val_v[pl.ds(g * 16, 16)]
        plsc.addupdate_scatter(rows_v, [rows, cols], vals)
    pltpu.sync_copy(rows_v, patch.at[pl.ds(wid * _EPT, _EPT)])


def _tc_body(y_s, lo_s, hi_s, P_ref, Yb_ref, out_ref):
    b = pl.program_id(0)
    rb = pl.program_id(1)
    out_ref[...] = Yb_ref[...]

    def upd(k, carry):
        local = y_s[k] - rb * _RB
        out_ref[0, pl.ds(local, 1), :] = P_ref[pl.ds(b * _K + k, 1), :]
        return carry

    jax.lax.fori_loop(lo_s[rb], hi_s[rb], upd, 0)


def kernel(Y, X, y_idx, x_idx):
    bb = jnp.repeat(jnp.arange(_B, dtype=jnp.int32), _K)
    yk = jnp.tile(y_idx, (_B,))
    xk = jnp.tile(x_idx, (_B,))
    row_e = (bb * _H + yk).reshape(_NW, _EPT)
    off_e = xk.reshape(_NW, _EPT)
    val_e = X.reshape(_NW, _EPT)

    patch = _sc_patch(Y.reshape(_B * _H, _W), row_e, off_e, val_e)

    edges = jnp.arange(_NRB, dtype=jnp.int32) * _RB
    lo = jnp.searchsorted(y_idx, edges).astype(jnp.int32)
    hi = jnp.searchsorted(y_idx, edges + _RB).astype(jnp.int32)

    out = pl.pallas_call(
        _tc_body,
        grid_spec=pltpu.PrefetchScalarGridSpec(
            num_scalar_prefetch=3,
            grid=(_B, _NRB),
            in_specs=[
                pl.BlockSpec((_B * _K, _W), lambda b, rb, *_: (0, 0)),
                pl.BlockSpec((1, _RB, _W), lambda b, rb, *_: (b, rb, 0)),
            ],
            out_specs=pl.BlockSpec((1, _RB, _W), lambda b, rb, *_: (b, rb, 0)),
        ),
        out_shape=jax.ShapeDtypeStruct((_B, _H, _W), jnp.float32),
    )(y_idx, lo, hi, patch, Y)
    return out


# speedup vs baseline: 48.0772x; 1.1831x over previous
import jax
import jax.numpy as jnp
from jax.experimental import pallas as pl
from jax.experimental.pallas import tpu as pltpu

_B, _H, _W = 16, 2048, 2048
_K = 64
_RB = 512
_NRB = _H // _RB


def _body(y_s, x_s, lo_s, hi_s, X_s, Yb_ref, out_ref):
    b = pl.program_id(0)
    rb = pl.program_id(1)
    out_ref[...] = Yb_ref[...]
    lane = jax.lax.broadcasted_iota(jnp.int32, (1, _W), 1)

    def upd(k, carry):
        y = y_s[k]
        col = x_s[k]
        val = X_s[b, k]
        local = y - rb * _RB
        row = Yb_ref[0, pl.ds(local, 1), :]
        out_ref[0, pl.ds(local, 1), :] = row + jnp.where(lane == col, val, 0.0)
        return carry

    jax.lax.fori_loop(lo_s[rb], hi_s[rb], upd, 0)


def kernel(Y, X, y_idx, x_idx):
    edges = jnp.arange(_NRB, dtype=jnp.int32) * _RB
    lo = jnp.searchsorted(y_idx, edges).astype(jnp.int32)
    hi = jnp.searchsorted(y_idx, edges + _RB).astype(jnp.int32)
    out = pl.pallas_call(
        _body,
        grid_spec=pltpu.PrefetchScalarGridSpec(
            num_scalar_prefetch=5,
            grid=(_B, _NRB),
            in_specs=[
                pl.BlockSpec((1, _RB, _W), lambda b, rb, *_: (b, rb, 0)),
            ],
            out_specs=pl.BlockSpec((1, _RB, _W), lambda b, rb, *_: (b, rb, 0)),
        ),
        out_shape=jax.ShapeDtypeStruct((_B, _H, _W), jnp.float32),
    )(y_idx, x_idx, lo, hi, X, Y)
    return out


# speedup vs baseline: 48.6825x; 1.0126x over previous
import jax
import jax.numpy as jnp
from jax.experimental import pallas as pl
from jax.experimental.pallas import tpu as pltpu

_B, _H, _W = 16, 2048, 2048
_K = 64
_RB = 1024
_NRB = _H // _RB


def _body(y_s, x_s, lo_s, hi_s, X_s, Yb_ref, out_ref):
    b = pl.program_id(0)
    rb = pl.program_id(1)
    out_ref[...] = Yb_ref[...]
    lane = jax.lax.broadcasted_iota(jnp.int32, (1, _W), 1)

    def upd(k, carry):
        y = y_s[k]
        col = x_s[k]
        val = X_s[b, k]
        local = y - rb * _RB
        row = Yb_ref[0, pl.ds(local, 1), :]
        out_ref[0, pl.ds(local, 1), :] = row + jnp.where(lane == col, val, 0.0)
        return carry

    jax.lax.fori_loop(lo_s[rb], hi_s[rb], upd, 0)


def kernel(Y, X, y_idx, x_idx):
    edges = jnp.arange(_NRB, dtype=jnp.int32) * _RB
    lo = jnp.searchsorted(y_idx, edges).astype(jnp.int32)
    hi = jnp.searchsorted(y_idx, edges + _RB).astype(jnp.int32)
    out = pl.pallas_call(
        _body,
        grid_spec=pltpu.PrefetchScalarGridSpec(
            num_scalar_prefetch=5,
            grid=(_B, _NRB),
            in_specs=[
                pl.BlockSpec((1, _RB, _W), lambda b, rb, *_: (b, rb, 0)),
            ],
            out_specs=pl.BlockSpec((1, _RB, _W), lambda b, rb, *_: (b, rb, 0)),
        ),
        out_shape=jax.ShapeDtypeStruct((_B, _H, _W), jnp.float32),
    )(y_idx, x_idx, lo, hi, X, Y)
    return out
